# Initial kernel scaffold; baseline (speedup 1.0000x reference)
import jax
import jax.numpy as jnp
from jax.experimental import pallas as pl


def kernel(input, target):
    raise NotImplementedError("write your pallas kernel here")


# speedup vs baseline: 54.6362x; 54.6362x over previous
import numpy as np
import jax
import jax.numpy as jnp
from jax.experimental import pallas as pl
from jax.experimental.pallas import tpu as pltpu

_FST = (1.0, 1.0, 1.0)


def _base_index(pos: int) -> np.ndarray:
    idx = np.zeros(66, np.int32)
    for k in range(64):
        bases = (k // 16, (k // 4) % 4, k % 4)
        idx[k + 1] = bases[pos - 1] + 1
    idx[65] = 5
    return idx


def _w_matrix() -> np.ndarray:
    W = np.zeros((24, 66), np.float32)
    for p in range(3):
        bidx = _base_index(p + 1)
        for c in range(6):
            W[8 * p + c, c] += 1.0
        for j in range(66):
            W[8 * p + int(bidx[j]), j] += 1.0
    return W


def _loss_body(x_ref, t_ref, w_ref, o_ref):
    b = pl.program_id(0)
    x = x_ref[0]
    t = t_ref[0]
    W = w_ref[...]

    A = jnp.dot(W, x, preferred_element_type=jnp.float32)
    xt = x[6:66]
    M0 = jnp.max(xt, axis=0, keepdims=True)
    S0 = jnp.sum(jnp.exp(xt - M0), axis=0, keepdims=True)

    u = t - 1
    total = jnp.zeros_like(M0)
    for p in range(3):
        Ap = A[8 * p:8 * p + 6]
        Mp = jnp.maximum(M0, jnp.max(Ap, axis=0, keepdims=True))
        Tp = S0 * jnp.exp(M0 - Mp) + jnp.sum(jnp.exp(Ap - Mp), axis=0,
                                             keepdims=True)
        if p == 0:
            d = u // 16
        elif p == 1:
            d = (u // 4) % 4
        else:
            d = u % 4
        bt = jnp.where(t == 0, 0, jnp.where(t == 65, 5, d + 1))
        picked = jnp.zeros_like(M0)
        for m in range(6):
            picked += jnp.where(bt == m, Ap[m:m + 1], 0.0)
        total += (jnp.log(Tp) + Mp - picked) * _FST[p]

    @pl.when(b == 0)
    def _init():
        o_ref[0, 0] = 0.0

    o_ref[0, 0] += jnp.sum(total)


def kernel(input, target):
    B, C, L = input.shape
    t = target.astype(jnp.int32).reshape(B, 1, L)
    W = jnp.asarray(_w_matrix())
    out = pl.pallas_call(
        _loss_body,
        grid=(B,),
        in_specs=[
            pl.BlockSpec((1, C, L), lambda b: (b, 0, 0)),
            pl.BlockSpec((1, 1, L), lambda b: (b, 0, 0)),
            pl.BlockSpec((24, 66), lambda b: (0, 0)),
        ],
        out_specs=pl.BlockSpec(memory_space=pltpu.SMEM),
        out_shape=jax.ShapeDtypeStruct((1, 1), jnp.float32),
        compiler_params=pltpu.CompilerParams(
            dimension_semantics=("arbitrary",)),
    )(input, t, W)
    return out[0, 0] / jnp.float32(B * L)
